# baseline (speedup 1.0000x reference)
import functools

import jax
import jax.numpy as jnp
from jax import lax
from jax.experimental import pallas as pl
from jax.experimental.pallas import tpu as pltpu
from jax.experimental.pallas import tpu_sc as plsc

N = 1024
E = 65536
H = 6
D = 64
HID = H * D

_ROWS = N // 16
_EHALF = E // 2
_CHUNK = 8192
_RT = 256
_F32 = jnp.float32
_PREC = lax.Precision.HIGHEST
_PREC_AGG = lax.Precision.DEFAULT


def _count_body(g_hbm, zeros_hbm, out_hbm, cmat, srcb, dstb):
    c = lax.axis_index("c")
    s = lax.axis_index("s")
    base = s * _ROWS
    pltpu.sync_copy(zeros_hbm, cmat)
    e0 = c * _EHALF
    ones = jnp.ones((16,), _F32)

    def chunk(ci, carry):
        off = e0 + ci * _CHUNK
        pltpu.sync_copy(g_hbm.at[0, pl.ds(off, _CHUNK)], srcb)
        pltpu.sync_copy(g_hbm.at[1, pl.ds(off, _CHUNK)], dstb)

        def step(j, carry2):
            for u in range(4):
                d = dstb[pl.ds(j * 64 + u * 16, 16)]
                sv = srcb[pl.ds(j * 64 + u * 16, 16)]
                rel = d - base
                m = (rel >= 0) & (rel < _ROWS)
                relc = jnp.where(m, rel, 0)
                flat = relc * N + sv
                plsc.addupdate_scatter(cmat, [flat], ones, mask=m)
            return carry2

        return lax.fori_loop(0, _CHUNK // 64, step, carry)

    lax.fori_loop(0, _EHALF // _CHUNK, chunk, 0)
    pltpu.sync_copy(cmat, out_hbm.at[c, s])


def _build_counts(g, zeros):
    call = pl.kernel(
        _count_body,
        out_type=jax.ShapeDtypeStruct((2, 16, _ROWS * N), _F32),
        mesh=plsc.VectorSubcoreMesh(core_axis_name="c", subcore_axis_name="s"),
        compiler_params=pltpu.CompilerParams(needs_layout_passes=False),
        scratch_types=[
            pltpu.VMEM((_ROWS * N,), _F32),
            pltpu.VMEM((_CHUNK,), jnp.int32),
            pltpu.VMEM((_CHUNK,), jnp.int32),
        ],
    )
    return call(g, zeros).reshape(2, N, N)


def _fc_body(x_ref, w_ref, al_ref, ar_ref, z_ref, el_ref, er_ref):
    z = jnp.dot(x_ref[...], w_ref[...], preferred_element_type=_F32,
                precision=_PREC)
    z_ref[...] = z
    els, ers = [], []
    for h in range(H):
        zh = z[:, h * D:(h + 1) * D]
        els.append(jnp.sum(zh * al_ref[:, h * D:(h + 1) * D], axis=1, keepdims=True))
        ers.append(jnp.sum(zh * ar_ref[:, h * D:(h + 1) * D], axis=1, keepdims=True))
    el_ref[...] = jnp.concatenate(els, axis=1)
    er_ref[...] = jnp.concatenate(ers, axis=1)


def _fc_call(x, w, al, ar):
    k = x.shape[1]
    return pl.pallas_call(
        _fc_body,
        grid=(N // _RT,),
        in_specs=[
            pl.BlockSpec((_RT, k), lambda i: (i, 0)),
            pl.BlockSpec((k, HID), lambda i: (0, 0)),
            pl.BlockSpec((1, HID), lambda i: (0, 0)),
            pl.BlockSpec((1, HID), lambda i: (0, 0)),
        ],
        out_specs=[
            pl.BlockSpec((_RT, HID), lambda i: (i, 0)),
            pl.BlockSpec((_RT, H), lambda i: (i, 0)),
            pl.BlockSpec((_RT, H), lambda i: (i, 0)),
        ],
        out_shape=[
            jax.ShapeDtypeStruct((N, HID), _F32),
            jax.ShapeDtypeStruct((N, H), _F32),
            jax.ShapeDtypeStruct((N, H), _F32),
        ],
    )(x, w, al, ar)


def _attn_body(*refs, first, last):
    if first:
        c0_ref, c1_ref, elt_ref, er_ref, feat_ref, b_ref, out_ref, cm_ref = refs
        cm = c0_ref[...] + c1_ref[...]
        cm_ref[...] = cm
    else:
        cin_ref, elt_ref, er_ref, feat_ref, b_ref, out_ref = refs
        cm = cin_ref[...]
    pos = cm > 0.0
    acc = None
    outs = []
    for h in range(H):
        el_h = elt_ref[h]
        er_h = er_ref[:, h:h + 1]
        ml = jnp.max(el_h)
        x_row = jnp.exp(el_h - ml)[None, :]
        u_row = jnp.exp(0.2 * (el_h - ml))[None, :]
        t = er_h + el_h[None, :]
        tm = jnp.max(jnp.where(pos, t, -1e30), axis=1, keepdims=True)
        em = jnp.where(tm > -1e29, jnp.maximum(tm, 0.2 * tm), 0.0)
        ky1 = jnp.exp(jnp.minimum(er_h + (ml - em), 60.0))
        ky2 = jnp.exp(jnp.minimum(0.2 * (er_h + ml) - em, 60.0))
        p = cm * jnp.where(t > 0.0, x_row * ky1, u_row * ky2)
        dn = jnp.sum(p, axis=1, keepdims=True)
        oh = jnp.dot(p, feat_ref[:, h * D:(h + 1) * D],
                     preferred_element_type=_F32, precision=_PREC_AGG)
        oh = oh * (1.0 / (dn + 1e-12))
        if last:
            acc = oh if acc is None else acc + oh
        else:
            outs.append(oh)
    if last:
        out_ref[...] = acc * (1.0 / H) + b_ref[...]
    else:
        o = jnp.concatenate(outs, axis=1) + b_ref[...]
        out_ref[...] = jnp.where(o > 0.0, o, jnp.exp(jnp.minimum(o, 0.0)) - 1.0)


def _attn_call(cparts, elt, er, feat, b, first, last):
    od = D if last else HID
    body = functools.partial(_attn_body, first=first, last=last)
    cspecs = [pl.BlockSpec((_RT, N), lambda i: (i, 0))] * (2 if first else 1)
    out_specs = pl.BlockSpec((_RT, od), lambda i: (i, 0))
    out_shape = jax.ShapeDtypeStruct((N, od), _F32)
    if first:
        out_specs = [out_specs, pl.BlockSpec((_RT, N), lambda i: (i, 0))]
        out_shape = [out_shape, jax.ShapeDtypeStruct((N, N), _F32)]
    return pl.pallas_call(
        body,
        grid=(N // _RT,),
        in_specs=cspecs + [
            pl.BlockSpec((H, N), lambda i: (0, 0)),
            pl.BlockSpec((_RT, H), lambda i: (i, 0)),
            pl.BlockSpec((N, HID), lambda i: (0, 0)),
            pl.BlockSpec((1, od), lambda i: (0, 0)),
        ],
        out_specs=out_specs,
        out_shape=out_shape,
    )(*cparts, elt, er, feat, b)


def _od_body(dis_ref, emb_ref, embt_ref, wp1_ref, wp2_ref, sc_ref, od_ref):
    lin1 = jnp.dot(wp1_ref[...], embt_ref[...], preferred_element_type=_F32,
                   precision=_PREC)
    lin2 = jnp.sum(emb_ref[...] * wp2_ref[...], axis=1, keepdims=True)
    od_ref[...] = jnp.tanh(lin2 + lin1 + dis_ref[...] * sc_ref[:, 0:1]
                           + sc_ref[:, 1:2])


def _od_call(dis, emb, embt, wp1, wp2, sc):
    return pl.pallas_call(
        _od_body,
        grid=(N // _RT,),
        in_specs=[
            pl.BlockSpec((_RT, N), lambda i: (i, 0)),
            pl.BlockSpec((_RT, D), lambda i: (i, 0)),
            pl.BlockSpec((D, N), lambda i: (0, 0)),
            pl.BlockSpec((1, D), lambda i: (0, 0)),
            pl.BlockSpec((1, D), lambda i: (0, 0)),
            pl.BlockSpec((1, 2), lambda i: (0, 0)),
        ],
        out_specs=pl.BlockSpec((_RT, N), lambda i: (i, 0)),
        out_shape=jax.ShapeDtypeStruct((N, N), _F32),
    )(dis, emb, embt, wp1, wp2, sc)


def kernel(nfeats, g, dis, params):
    cparts = _build_counts(g, jnp.zeros((_ROWS * N,), _F32))

    h = jnp.pad(nfeats, ((0, 0), (0, 256 - nfeats.shape[1])))
    cm = None
    for l in range(4):
        w = params[f"W{l}"]
        if l == 0:
            w = jnp.pad(w, ((0, 256 - w.shape[0]), (0, 0)))
        al = params[f"al{l}"].reshape(1, HID)
        ar = params[f"ar{l}"].reshape(1, HID)
        z, el, er = _fc_call(h, w, al, ar)
        first = l == 0
        last = l == 3
        b = params[f"b{l}"]
        bb = b.reshape(H, D).mean(axis=0)[None, :] if last else b[None, :]
        cin = [cparts[0], cparts[1]] if first else [cm]
        h = _attn_call(cin, el.T, er, z, bb, first, last)
        if first:
            h, cm = h

    emb = h
    wp = params["Wp"][:, 0]
    sc = jnp.stack([wp[128], params["bp"][0]]).reshape(1, 2)
    return _od_call(dis, emb, emb.T, wp[:64][None, :], wp[64:128][None, :], sc)


# speedup vs baseline: 1.2063x; 1.2063x over previous
import functools

import jax
import jax.numpy as jnp
from jax import lax
from jax.experimental import pallas as pl
from jax.experimental.pallas import tpu as pltpu
from jax.experimental.pallas import tpu_sc as plsc

N = 1024
E = 65536
H = 6
D = 64
HID = H * D

_ROWS = N // 16
_EHALF = E // 2
_CHUNK = 8192
_RT = 256
_F32 = jnp.float32
_PREC = lax.Precision.HIGHEST
_PREC_AGG = lax.Precision.DEFAULT


def _count_body(g_hbm, zeros_hbm, out_hbm, cmat, srcb, dstb):
    c = lax.axis_index("c")
    s = lax.axis_index("s")
    base = s * _ROWS
    pltpu.sync_copy(zeros_hbm, cmat)
    e0 = c * _EHALF
    ones = jnp.ones((16,), _F32)

    def chunk(ci, carry):
        off = e0 + ci * _CHUNK
        pltpu.sync_copy(g_hbm.at[0, pl.ds(off, _CHUNK)], srcb)
        pltpu.sync_copy(g_hbm.at[1, pl.ds(off, _CHUNK)], dstb)

        def step(j, carry2):
            for u in range(4):
                d = dstb[pl.ds(j * 64 + u * 16, 16)]
                sv = srcb[pl.ds(j * 64 + u * 16, 16)]
                rel = d - base
                m = (rel >= 0) & (rel < _ROWS)
                relc = jnp.where(m, rel, 0)
                flat = relc * N + sv
                plsc.addupdate_scatter(cmat, [flat], ones, mask=m)
            return carry2

        return lax.fori_loop(0, _CHUNK // 64, step, carry)

    lax.fori_loop(0, _EHALF // _CHUNK, chunk, 0)
    pltpu.sync_copy(cmat, out_hbm.at[c, s])


def _build_counts(g, zeros):
    call = pl.kernel(
        _count_body,
        out_type=jax.ShapeDtypeStruct((2, 16, _ROWS * N), _F32),
        mesh=plsc.VectorSubcoreMesh(core_axis_name="c", subcore_axis_name="s"),
        compiler_params=pltpu.CompilerParams(needs_layout_passes=False),
        scratch_types=[
            pltpu.VMEM((_ROWS * N,), _F32),
            pltpu.VMEM((_CHUNK,), jnp.int32),
            pltpu.VMEM((_CHUNK,), jnp.int32),
        ],
    )
    return call(g, zeros).reshape(2, N, N)


def _layer_body(*refs, first, last):
    if first:
        (x_ref, w_ref, al_ref, ar_ref, c0_ref, c1_ref, b_ref,
         out_ref, cm_out_ref, feat_s, el_s, er_s, elt_s) = refs
    elif last:
        (x_ref, w_ref, al_ref, ar_ref, cin_ref, b_ref,
         dis_ref, wp1_ref, wp2_ref, sc_ref,
         od_ref, feat_s, el_s, er_s, elt_s, emb_s, embt_s) = refs
    else:
        (x_ref, w_ref, al_ref, ar_ref, cin_ref, b_ref,
         out_ref, feat_s, el_s, er_s, elt_s) = refs

    i = pl.program_id(0)

    @pl.when(i < 4)
    def _fc_phase():
        z = jnp.dot(x_ref[...], w_ref[...], preferred_element_type=_F32,
                    precision=_PREC)
        r0 = i * _RT
        feat_s[pl.ds(r0, _RT), :] = z
        els, ers = [], []
        for h in range(H):
            zh = z[:, h * D:(h + 1) * D]
            els.append(jnp.sum(zh * al_ref[:, h * D:(h + 1) * D],
                               axis=1, keepdims=True))
            ers.append(jnp.sum(zh * ar_ref[:, h * D:(h + 1) * D],
                               axis=1, keepdims=True))
        el_s[pl.ds(r0, _RT), :] = jnp.concatenate(els, axis=1)
        er_s[pl.ds(r0, _RT), :] = jnp.concatenate(ers, axis=1)

    @pl.when(i == 4)
    def _mk_elt():
        elt_s[...] = el_s[...].T

    @pl.when((i >= 4) & (i < 8))
    def _attn_phase():
        r0 = (i - 4) * _RT
        if first:
            cm = c0_ref[0] + c1_ref[0]
            cm_out_ref[...] = cm
        else:
            cm = cin_ref[...]
        pos = cm > 0.0
        er_t = er_s[pl.ds(r0, _RT), :]
        acc = None
        outs = []
        for h in range(H):
            el_h = elt_s[h]
            t = er_t[:, h:h + 1] + el_h[None, :]
            e = jnp.maximum(t, 0.2 * t)
            em = jnp.max(jnp.where(pos, e, -1e30), axis=1, keepdims=True)
            p = cm * jnp.exp(jnp.minimum(e - em, 0.0))
            dn = jnp.sum(p, axis=1, keepdims=True)
            oh = jnp.dot(p, feat_s[:, h * D:(h + 1) * D],
                         preferred_element_type=_F32, precision=_PREC_AGG)
            oh = oh * (1.0 / (dn + 1e-12))
            if last:
                acc = oh if acc is None else acc + oh
            else:
                outs.append(oh)
        if last:
            emb_s[pl.ds(r0, _RT), :] = acc * (1.0 / H) + b_ref[...]
        else:
            o = jnp.concatenate(outs, axis=1) + b_ref[...]
            out_ref[...] = jnp.where(o > 0.0,
                                     o, jnp.exp(jnp.minimum(o, 0.0)) - 1.0)

    if last:
        @pl.when(i == 8)
        def _mk_embt():
            embt_s[...] = emb_s[...].T

        @pl.when(i >= 8)
        def _od_phase():
            r0 = (i - 8) * _RT
            lin1 = jnp.dot(wp1_ref[...], embt_s[...],
                           preferred_element_type=_F32, precision=_PREC)
            lin2 = jnp.sum(emb_s[pl.ds(r0, _RT), :] * wp2_ref[...],
                           axis=1, keepdims=True)
            od_ref[...] = jnp.tanh(lin2 + lin1 + dis_ref[...] * sc_ref[:, 0:1]
                                   + sc_ref[:, 1:2])


def _layer_call(x, w, al, ar, cin, b, first, last, extra=None):
    k = x.shape[1]
    body = functools.partial(_layer_body, first=first, last=last)
    nsteps = 12 if last else 8

    def xmap(i):
        return (jnp.minimum(i, 3), 0)

    def amap(i):
        return (jnp.clip(i - 4, 0, 3), 0)

    in_specs = [
        pl.BlockSpec((_RT, k), xmap),
        pl.BlockSpec((k, HID), lambda i: (0, 0)),
        pl.BlockSpec((1, HID), lambda i: (0, 0)),
        pl.BlockSpec((1, HID), lambda i: (0, 0)),
    ]
    if first:
        in_specs += [
            pl.BlockSpec((1, _RT, N), lambda i: (0, jnp.clip(i - 4, 0, 3), 0)),
            pl.BlockSpec((1, _RT, N), lambda i: (1, jnp.clip(i - 4, 0, 3), 0)),
        ]
        operands = [x, w, al, ar, cin, cin, b]
    else:
        in_specs += [pl.BlockSpec((_RT, N), amap)]
        operands = [x, w, al, ar, cin, b]
    in_specs += [pl.BlockSpec((1, D if last else HID), lambda i: (0, 0))]

    scratch = [
        pltpu.VMEM((N, HID), _F32),
        pltpu.VMEM((N, H), _F32),
        pltpu.VMEM((N, H), _F32),
        pltpu.VMEM((H, N), _F32),
    ]
    if last:
        dis, wp1, wp2, sc = extra
        in_specs += [
            pl.BlockSpec((_RT, N), lambda i: (jnp.clip(i - 8, 0, 3), 0)),
            pl.BlockSpec((1, D), lambda i: (0, 0)),
            pl.BlockSpec((1, D), lambda i: (0, 0)),
            pl.BlockSpec((1, 2), lambda i: (0, 0)),
        ]
        operands += [dis, wp1, wp2, sc]
        out_specs = pl.BlockSpec((_RT, N), lambda i: (jnp.clip(i - 8, 0, 3), 0))
        out_shape = jax.ShapeDtypeStruct((N, N), _F32)
        scratch += [pltpu.VMEM((N, D), _F32), pltpu.VMEM((D, N), _F32)]
    elif first:
        out_specs = [pl.BlockSpec((_RT, HID), amap), pl.BlockSpec((_RT, N), amap)]
        out_shape = [jax.ShapeDtypeStruct((N, HID), _F32),
                     jax.ShapeDtypeStruct((N, N), _F32)]
    else:
        out_specs = pl.BlockSpec((_RT, HID), amap)
        out_shape = jax.ShapeDtypeStruct((N, HID), _F32)

    return pl.pallas_call(
        body,
        grid=(nsteps,),
        in_specs=in_specs,
        out_specs=out_specs,
        out_shape=out_shape,
        scratch_shapes=scratch,
    )(*operands)


def kernel(nfeats, g, dis, params):
    cparts = _build_counts(g, jnp.zeros((_ROWS * N,), _F32))

    h = nfeats
    cm = None
    out = None
    for l in range(4):
        w = params[f"W{l}"]
        al = params[f"al{l}"].reshape(1, HID)
        ar = params[f"ar{l}"].reshape(1, HID)
        first = l == 0
        last = l == 3
        b = params[f"b{l}"]
        bb = b.reshape(H, D).mean(axis=0)[None, :] if last else b[None, :]
        if last:
            wp = params["Wp"][:, 0]
            sc = jnp.stack([wp[128], params["bp"][0]]).reshape(1, 2)
            extra = (dis, wp[:64][None, :], wp[64:128][None, :], sc)
            out = _layer_call(h, w, al, ar, cm, bb, first, last, extra)
        elif first:
            h, cm = _layer_call(h, w, al, ar, cparts, bb, first, last)
        else:
            h = _layer_call(h, w, al, ar, cm, bb, first, last)
    return out


# speedup vs baseline: 1.3172x; 1.0919x over previous
import functools

import jax
import jax.numpy as jnp
from jax import lax
from jax.experimental import pallas as pl
from jax.experimental.pallas import tpu as pltpu
from jax.experimental.pallas import tpu_sc as plsc

N = 1024
E = 65536
H = 6
D = 64
HID = H * D

_ROWS = N // 16
_EHALF = E // 2
_CHUNK = 8192
_RT = 256
_F32 = jnp.float32
_PREC = lax.Precision.HIGHEST
_PREC_AGG = lax.Precision.DEFAULT


def _count_body(g_hbm, zeros_hbm, out_hbm, cmat, srcb, dstb):
    c = lax.axis_index("c")
    s = lax.axis_index("s")
    base = s * _ROWS
    pltpu.sync_copy(zeros_hbm, cmat)
    e0 = c * _EHALF
    ones = jnp.ones((16,), _F32)

    def chunk(ci, carry):
        off = e0 + ci * _CHUNK
        pltpu.sync_copy(g_hbm.at[0, pl.ds(off, _CHUNK)], srcb)
        pltpu.sync_copy(g_hbm.at[1, pl.ds(off, _CHUNK)], dstb)

        def step(j, carry2):
            for u in range(4):
                d = dstb[pl.ds(j * 64 + u * 16, 16)]
                sv = srcb[pl.ds(j * 64 + u * 16, 16)]
                rel = d - base
                m = (rel >= 0) & (rel < _ROWS)
                relc = jnp.where(m, rel, 0)
                plsc.addupdate_scatter(cmat, [relc, sv], ones, mask=m)
            return carry2

        return lax.fori_loop(0, _CHUNK // 64, step, carry)

    lax.fori_loop(0, _EHALF // _CHUNK, chunk, 0)
    pltpu.sync_copy(cmat, out_hbm.at[c, pl.ds(base, _ROWS), :])


def _build_counts(g, zeros):
    call = pl.kernel(
        _count_body,
        out_type=jax.ShapeDtypeStruct((2, N, N), _F32),
        mesh=plsc.VectorSubcoreMesh(core_axis_name="c", subcore_axis_name="s"),
        compiler_params=pltpu.CompilerParams(needs_layout_passes=False),
        scratch_types=[
            pltpu.VMEM((_ROWS, N), _F32),
            pltpu.VMEM((_CHUNK,), jnp.int32),
            pltpu.VMEM((_CHUNK,), jnp.int32),
        ],
    )
    return call(g, zeros)


def _fc_compute(x_ref, w_ref, al_ref, ar_ref):
    z = jnp.dot(x_ref[...], w_ref[...], preferred_element_type=_F32,
                precision=_PREC)
    els, ers = [], []
    for h in range(H):
        zh = z[:, h * D:(h + 1) * D]
        els.append(jnp.sum(zh * al_ref[h][None, :], axis=1, keepdims=True))
        ers.append(jnp.sum(zh * ar_ref[h][None, :], axis=1, keepdims=True))
    return z, jnp.concatenate(els, axis=1), jnp.concatenate(ers, axis=1)


def _attn_compute(cm, elt_s, er_t, feat_s, last):
    acc = None
    outs = []
    for h in range(H):
        el_h = elt_s[h]
        t = er_t[:, h:h + 1] + el_h[None, :]
        e = jnp.maximum(t, 0.2 * t)
        em = jnp.max(e, axis=1, keepdims=True)
        p = cm * jnp.exp(e - em)
        dn = jnp.sum(p, axis=1, keepdims=True)
        oh = jnp.dot(p, feat_s[:, h * D:(h + 1) * D],
                     preferred_element_type=_F32, precision=_PREC_AGG)
        oh = oh * (1.0 / (dn + 1e-12))
        if last:
            acc = oh if acc is None else acc + oh
        else:
            outs.append(oh)
    return acc if last else outs


def _fc0_body(x_ref, w_ref, al_ref, ar_ref, z_ref, el_ref, er_ref):
    z, el, er = _fc_compute(x_ref, w_ref, al_ref, ar_ref)
    z_ref[...] = z
    el_ref[...] = el
    er_ref[...] = er


def _fc0_call(x, w, al, ar):
    k = x.shape[1]
    return pl.pallas_call(
        _fc0_body,
        grid=(N // _RT,),
        in_specs=[
            pl.BlockSpec((_RT, k), lambda i: (i, 0)),
            pl.BlockSpec((k, HID), lambda i: (0, 0)),
            pl.BlockSpec((H, D), lambda i: (0, 0)),
            pl.BlockSpec((H, D), lambda i: (0, 0)),
        ],
        out_specs=[
            pl.BlockSpec((_RT, HID), lambda i: (i, 0)),
            pl.BlockSpec((_RT, H), lambda i: (i, 0)),
            pl.BlockSpec((_RT, H), lambda i: (i, 0)),
        ],
        out_shape=[
            jax.ShapeDtypeStruct((N, HID), _F32),
            jax.ShapeDtypeStruct((N, H), _F32),
            jax.ShapeDtypeStruct((N, H), _F32),
        ],
    )(x, w, al, ar)


def _attn0_body(c0_ref, c1_ref, el_ref, er_ref, feat_ref, b_ref,
                out_ref, cm_out_ref, elt_s):
    i = pl.program_id(0)

    @pl.when(i == 0)
    def _mk_elt():
        elt_s[...] = el_ref[...].T

    cm = c0_ref[0] + c1_ref[0]
    cm_out_ref[...] = cm
    outs = _attn_compute(cm, elt_s, er_ref[...], feat_ref, last=False)
    o = jnp.concatenate(outs, axis=1) + b_ref[...][None, :]
    out_ref[...] = jnp.where(o > 0.0, o, jnp.exp(jnp.minimum(o, 0.0)) - 1.0)


def _attn0_call(cparts, el, er, feat, b):
    return pl.pallas_call(
        _attn0_body,
        grid=(N // _RT,),
        in_specs=[
            pl.BlockSpec((1, _RT, N), lambda i: (0, i, 0)),
            pl.BlockSpec((1, _RT, N), lambda i: (1, i, 0)),
            pl.BlockSpec((N, H), lambda i: (0, 0)),
            pl.BlockSpec((_RT, H), lambda i: (i, 0)),
            pl.BlockSpec((N, HID), lambda i: (0, 0)),
            pl.BlockSpec((HID,), lambda i: (0,)),
        ],
        out_specs=[
            pl.BlockSpec((_RT, HID), lambda i: (i, 0)),
            pl.BlockSpec((_RT, N), lambda i: (i, 0)),
        ],
        out_shape=[
            jax.ShapeDtypeStruct((N, HID), _F32),
            jax.ShapeDtypeStruct((N, N), _F32),
        ],
        scratch_shapes=[pltpu.VMEM((H, N), _F32)],
    )(cparts, cparts, el, er, feat, b)


def _layer_body(*refs, last):
    if last:
        (x_ref, w_ref, al_ref, ar_ref, cin_ref, b_ref,
         dis_ref, wp1_ref, wp2_ref, sc_ref,
         od_ref, feat_s, el_s, er_s, elt_s, emb_s, embt_s) = refs
    else:
        (x_ref, w_ref, al_ref, ar_ref, cin_ref, b_ref,
         out_ref, feat_s, el_s, er_s, elt_s) = refs

    i = pl.program_id(0)

    @pl.when(i < 4)
    def _fc_phase():
        z, el, er = _fc_compute(x_ref, w_ref, al_ref, ar_ref)
        r0 = i * _RT
        feat_s[pl.ds(r0, _RT), :] = z
        el_s[pl.ds(r0, _RT), :] = el
        er_s[pl.ds(r0, _RT), :] = er

    @pl.when(i == 4)
    def _mk_elt():
        elt_s[...] = el_s[...].T

    @pl.when((i >= 4) & (i < 8))
    def _attn_phase():
        r0 = (i - 4) * _RT
        cm = cin_ref[...]
        er_t = er_s[pl.ds(r0, _RT), :]
        res = _attn_compute(cm, elt_s, er_t, feat_s, last)
        if last:
            bmean = sum(b_ref[h * D:(h + 1) * D][None, :]
                        for h in range(H)) * (1.0 / H)
            emb_s[pl.ds(r0, _RT), :] = res * (1.0 / H) + bmean
        else:
            o = jnp.concatenate(res, axis=1) + b_ref[...][None, :]
            out_ref[...] = jnp.where(o > 0.0,
                                     o, jnp.exp(jnp.minimum(o, 0.0)) - 1.0)

    if last:
        @pl.when(i == 8)
        def _mk_embt():
            embt_s[...] = emb_s[...].T

        @pl.when(i >= 8)
        def _od_phase():
            r0 = (i - 8) * _RT
            lin1 = jnp.dot(wp1_ref[...], embt_s[...],
                           preferred_element_type=_F32, precision=_PREC)
            lin2 = jnp.sum(emb_s[pl.ds(r0, _RT), :] * wp2_ref[...],
                           axis=1, keepdims=True)
            od_ref[...] = jnp.tanh(lin2 + lin1 + dis_ref[...] * sc_ref[:, 0:1]
                                   + sc_ref[:, 1:2])


def _layer_call(x, w, al, ar, cin, b, last, extra=None):
    body = functools.partial(_layer_body, last=last)
    nsteps = 12 if last else 8

    def amap(i):
        return (jnp.clip(i - 4, 0, 3), 0)

    in_specs = [
        pl.BlockSpec((_RT, HID), lambda i: (jnp.minimum(i, 3), 0)),
        pl.BlockSpec((HID, HID), lambda i: (0, 0)),
        pl.BlockSpec((H, D), lambda i: (0, 0)),
        pl.BlockSpec((H, D), lambda i: (0, 0)),
        pl.BlockSpec((_RT, N), amap),
        pl.BlockSpec((HID,), lambda i: (0,)),
    ]
    operands = [x, w, al, ar, cin, b]

    scratch = [
        pltpu.VMEM((N, HID), _F32),
        pltpu.VMEM((N, H), _F32),
        pltpu.VMEM((N, H), _F32),
        pltpu.VMEM((H, N), _F32),
    ]
    if last:
        dis, wp1, wp2, sc = extra
        in_specs += [
            pl.BlockSpec((_RT, N), lambda i: (jnp.clip(i - 8, 0, 3), 0)),
            pl.BlockSpec((1, D), lambda i: (0, 0)),
            pl.BlockSpec((1, D), lambda i: (0, 0)),
            pl.BlockSpec((1, 2), lambda i: (0, 0)),
        ]
        operands += [dis, wp1, wp2, sc]
        out_specs = pl.BlockSpec((_RT, N), lambda i: (jnp.clip(i - 8, 0, 3), 0))
        out_shape = jax.ShapeDtypeStruct((N, N), _F32)
        scratch += [pltpu.VMEM((N, D), _F32), pltpu.VMEM((D, N), _F32)]
    else:
        out_specs = pl.BlockSpec((_RT, HID), amap)
        out_shape = jax.ShapeDtypeStruct((N, HID), _F32)

    return pl.pallas_call(
        body,
        grid=(nsteps,),
        in_specs=in_specs,
        out_specs=out_specs,
        out_shape=out_shape,
        scratch_shapes=scratch,
    )(*operands)


def kernel(nfeats, g, dis, params):
    cparts = _build_counts(g, jnp.zeros((_ROWS, N), _F32))

    z0, el0, er0 = _fc0_call(nfeats, params["W0"], params["al0"], params["ar0"])
    h, cm = _attn0_call(cparts, el0, er0, z0, params["b0"])

    for l in (1, 2):
        h = _layer_call(h, params[f"W{l}"], params[f"al{l}"], params[f"ar{l}"],
                        cm, params[f"b{l}"], last=False)

    wp = params["Wp"][:, 0]
    sc = jnp.stack([wp[128], params["bp"][0]]).reshape(1, 2)
    extra = (dis, wp[:64][None, :], wp[64:128][None, :], sc)
    return _layer_call(h, params["W3"], params["al3"], params["ar3"],
                       cm, params["b3"], last=True, extra=extra)


# speedup vs baseline: 1.3772x; 1.0455x over previous
import functools

import jax
import jax.numpy as jnp
from jax import lax
from jax.experimental import pallas as pl
from jax.experimental.pallas import tpu as pltpu
from jax.experimental.pallas import tpu_sc as plsc

N = 1024
E = 65536
H = 6
D = 64
HID = H * D

_ROWS = N // 16
_EHALF = E // 2
_CHUNK = 8192
_RT = 512
_F32 = jnp.float32
_PREC = lax.Precision.HIGHEST
_PREC_AGG = lax.Precision.DEFAULT
_NT = N // _RT


def _count_body(g_hbm, zeros_hbm, out_hbm, cmat, srcb, dstb):
    c = lax.axis_index("c")
    s = lax.axis_index("s")
    base = s * _ROWS
    pltpu.sync_copy(zeros_hbm, cmat)
    e0 = c * _EHALF
    ones = jnp.ones((16,), _F32)

    def chunk(ci, carry):
        off = e0 + ci * _CHUNK
        pltpu.sync_copy(g_hbm.at[0, pl.ds(off, _CHUNK)], srcb)
        pltpu.sync_copy(g_hbm.at[1, pl.ds(off, _CHUNK)], dstb)

        def step(j, carry2):
            for u in range(4):
                d = dstb[pl.ds(j * 64 + u * 16, 16)]
                sv = srcb[pl.ds(j * 64 + u * 16, 16)]
                rel = d - base
                m = (rel >= 0) & (rel < _ROWS)
                relc = jnp.where(m, rel, 0)
                plsc.addupdate_scatter(cmat, [relc, sv], ones, mask=m)
            return carry2

        return lax.fori_loop(0, _CHUNK // 64, step, carry)

    lax.fori_loop(0, _EHALF // _CHUNK, chunk, 0)
    pltpu.sync_copy(cmat, out_hbm.at[c, pl.ds(base, _ROWS), :])


def _build_counts(g, zeros):
    call = pl.kernel(
        _count_body,
        out_type=jax.ShapeDtypeStruct((2, N, N), _F32),
        mesh=plsc.VectorSubcoreMesh(core_axis_name="c", subcore_axis_name="s"),
        compiler_params=pltpu.CompilerParams(needs_layout_passes=False),
        scratch_types=[
            pltpu.VMEM((_ROWS, N), _F32),
            pltpu.VMEM((_CHUNK,), jnp.int32),
            pltpu.VMEM((_CHUNK,), jnp.int32),
        ],
    )
    return call(g, zeros)


def _fc_compute(x_ref, w_ref, al_ref, ar_ref):
    z = jnp.dot(x_ref[...], w_ref[...], preferred_element_type=_F32,
                precision=_PREC)
    els, ers = [], []
    for h in range(H):
        zh = z[:, h * D:(h + 1) * D]
        els.append(jnp.sum(zh * al_ref[h][None, :], axis=1, keepdims=True))
        ers.append(jnp.sum(zh * ar_ref[h][None, :], axis=1, keepdims=True))
    return z, jnp.concatenate(els, axis=1), jnp.concatenate(ers, axis=1)


def _attn_compute(cm, elt_s, er_t, feat_s, last):
    acc = None
    outs = []
    for h in range(H):
        el_h = elt_s[h]
        t = er_t[:, h:h + 1] + el_h[None, :]
        e = jnp.maximum(t, 0.2 * t)
        em = jnp.max(e, axis=1, keepdims=True)
        p = cm * jnp.exp(e - em)
        dn = jnp.sum(p, axis=1, keepdims=True)
        oh = jnp.dot(p, feat_s[:, h * D:(h + 1) * D],
                     preferred_element_type=_F32, precision=_PREC_AGG)
        oh = oh * (1.0 / (dn + 1e-12))
        if last:
            acc = oh if acc is None else acc + oh
        else:
            outs.append(oh)
    return acc if last else outs


def _fc0_body(x_ref, w_ref, al_ref, ar_ref, z_ref, el_ref, er_ref):
    z, el, er = _fc_compute(x_ref, w_ref, al_ref, ar_ref)
    z_ref[...] = z
    el_ref[...] = el
    er_ref[...] = er


def _fc0_call(x, w, al, ar):
    k = x.shape[1]
    return pl.pallas_call(
        _fc0_body,
        grid=(N // _RT,),
        in_specs=[
            pl.BlockSpec((_RT, k), lambda i: (i, 0)),
            pl.BlockSpec((k, HID), lambda i: (0, 0)),
            pl.BlockSpec((H, D), lambda i: (0, 0)),
            pl.BlockSpec((H, D), lambda i: (0, 0)),
        ],
        out_specs=[
            pl.BlockSpec((_RT, HID), lambda i: (i, 0)),
            pl.BlockSpec((_RT, H), lambda i: (i, 0)),
            pl.BlockSpec((_RT, H), lambda i: (i, 0)),
        ],
        out_shape=[
            jax.ShapeDtypeStruct((N, HID), _F32),
            jax.ShapeDtypeStruct((N, H), _F32),
            jax.ShapeDtypeStruct((N, H), _F32),
        ],
    )(x, w, al, ar)


def _attn0_body(c0_ref, c1_ref, el_ref, er_ref, feat_ref, b_ref,
                out_ref, cm_out_ref, elt_s):
    i = pl.program_id(0)

    @pl.when(i == 0)
    def _mk_elt():
        elt_s[...] = el_ref[...].T

    cm = c0_ref[0] + c1_ref[0]
    cm_out_ref[...] = cm
    outs = _attn_compute(cm, elt_s, er_ref[...], feat_ref, last=False)
    o = jnp.concatenate(outs, axis=1) + b_ref[...][None, :]
    out_ref[...] = jnp.where(o > 0.0, o, jnp.exp(jnp.minimum(o, 0.0)) - 1.0)


def _attn0_call(cparts, el, er, feat, b):
    return pl.pallas_call(
        _attn0_body,
        grid=(N // _RT,),
        in_specs=[
            pl.BlockSpec((1, _RT, N), lambda i: (0, i, 0)),
            pl.BlockSpec((1, _RT, N), lambda i: (1, i, 0)),
            pl.BlockSpec((N, H), lambda i: (0, 0)),
            pl.BlockSpec((_RT, H), lambda i: (i, 0)),
            pl.BlockSpec((N, HID), lambda i: (0, 0)),
            pl.BlockSpec((HID,), lambda i: (0,)),
        ],
        out_specs=[
            pl.BlockSpec((_RT, HID), lambda i: (i, 0)),
            pl.BlockSpec((_RT, N), lambda i: (i, 0)),
        ],
        out_shape=[
            jax.ShapeDtypeStruct((N, HID), _F32),
            jax.ShapeDtypeStruct((N, N), _F32),
        ],
        scratch_shapes=[pltpu.VMEM((H, N), _F32)],
    )(cparts, cparts, el, er, feat, b)


def _layer_body(*refs, last):
    if last:
        (x_ref, w_ref, al_ref, ar_ref, cin_ref, b_ref,
         dis_ref, wp1_ref, wp2_ref, sc_ref,
         od_ref, feat_s, el_s, er_s, elt_s, emb_s, embt_s) = refs
    else:
        (x_ref, w_ref, al_ref, ar_ref, cin_ref, b_ref,
         out_ref, feat_s, el_s, er_s, elt_s) = refs

    i = pl.program_id(0)

    @pl.when(i < _NT)
    def _fc_phase():
        z, el, er = _fc_compute(x_ref, w_ref, al_ref, ar_ref)
        r0 = i * _RT
        feat_s[pl.ds(r0, _RT), :] = z
        el_s[pl.ds(r0, _RT), :] = el
        er_s[pl.ds(r0, _RT), :] = er

    @pl.when(i == _NT)
    def _mk_elt():
        elt_s[...] = el_s[...].T

    @pl.when((i >= _NT) & (i < 2 * _NT))
    def _attn_phase():
        r0 = (i - _NT) * _RT
        cm = cin_ref[...]
        er_t = er_s[pl.ds(r0, _RT), :]
        res = _attn_compute(cm, elt_s, er_t, feat_s, last)
        if last:
            bmean = sum(b_ref[h * D:(h + 1) * D][None, :]
                        for h in range(H)) * (1.0 / H)
            emb_s[pl.ds(r0, _RT), :] = res * (1.0 / H) + bmean
        else:
            o = jnp.concatenate(res, axis=1) + b_ref[...][None, :]
            out_ref[...] = jnp.where(o > 0.0,
                                     o, jnp.exp(jnp.minimum(o, 0.0)) - 1.0)

    if last:
        @pl.when(i == 2 * _NT)
        def _mk_embt():
            embt_s[...] = emb_s[...].T

        @pl.when(i >= 2 * _NT)
        def _od_phase():
            r0 = (i - 2 * _NT) * _RT
            lin1 = jnp.dot(wp1_ref[...], embt_s[...],
                           preferred_element_type=_F32, precision=_PREC)
            lin2 = jnp.sum(emb_s[pl.ds(r0, _RT), :] * wp2_ref[...],
                           axis=1, keepdims=True)
            od_ref[...] = jnp.tanh(lin2 + lin1 + dis_ref[...] * sc_ref[:, 0:1]
                                   + sc_ref[:, 1:2])


def _layer_call(x, w, al, ar, cin, b, last, extra=None):
    body = functools.partial(_layer_body, last=last)
    nsteps = 3 * _NT if last else 2 * _NT

    def amap(i):
        return (jnp.clip(i - _NT, 0, _NT - 1), 0)

    in_specs = [
        pl.BlockSpec((_RT, HID), lambda i: (jnp.minimum(i, _NT - 1), 0)),
        pl.BlockSpec((HID, HID), lambda i: (0, 0)),
        pl.BlockSpec((H, D), lambda i: (0, 0)),
        pl.BlockSpec((H, D), lambda i: (0, 0)),
        pl.BlockSpec((_RT, N), amap),
        pl.BlockSpec((HID,), lambda i: (0,)),
    ]
    operands = [x, w, al, ar, cin, b]

    scratch = [
        pltpu.VMEM((N, HID), _F32),
        pltpu.VMEM((N, H), _F32),
        pltpu.VMEM((N, H), _F32),
        pltpu.VMEM((H, N), _F32),
    ]
    if last:
        dis, wp1, wp2, sc = extra
        in_specs += [
            pl.BlockSpec((_RT, N), lambda i: (jnp.clip(i - 2 * _NT, 0, _NT - 1), 0)),
            pl.BlockSpec((1, D), lambda i: (0, 0)),
            pl.BlockSpec((1, D), lambda i: (0, 0)),
            pl.BlockSpec((1, 2), lambda i: (0, 0)),
        ]
        operands += [dis, wp1, wp2, sc]
        out_specs = pl.BlockSpec((_RT, N), lambda i: (jnp.clip(i - 2 * _NT, 0, _NT - 1), 0))
        out_shape = jax.ShapeDtypeStruct((N, N), _F32)
        scratch += [pltpu.VMEM((N, D), _F32), pltpu.VMEM((D, N), _F32)]
    else:
        out_specs = pl.BlockSpec((_RT, HID), amap)
        out_shape = jax.ShapeDtypeStruct((N, HID), _F32)

    return pl.pallas_call(
        body,
        grid=(nsteps,),
        in_specs=in_specs,
        out_specs=out_specs,
        out_shape=out_shape,
        scratch_shapes=scratch,
    )(*operands)


def kernel(nfeats, g, dis, params):
    cparts = _build_counts(g, jnp.zeros((_ROWS, N), _F32))

    z0, el0, er0 = _fc0_call(nfeats, params["W0"], params["al0"], params["ar0"])
    h, cm = _attn0_call(cparts, el0, er0, z0, params["b0"])

    for l in (1, 2):
        h = _layer_call(h, params[f"W{l}"], params[f"al{l}"], params[f"ar{l}"],
                        cm, params[f"b{l}"], last=False)

    wp = params["Wp"][:, 0]
    sc = jnp.stack([wp[128], params["bp"][0]]).reshape(1, 2)
    extra = (dis, wp[:64][None, :], wp[64:128][None, :], sc)
    return _layer_call(h, params["W3"], params["al3"], params["ar3"],
                       cm, params["b3"], last=True, extra=extra)
